# baseline (device time: 123403 ns/iter reference)
import jax
import jax.numpy as jnp
from jax import lax
from jax.experimental import pallas as pl
from jax.experimental.pallas import tpu as pltpu

N_DEV = 4
SQ = 2048
SKV = 2048
D_MODEL = 1024
HQ_PER = 8
DH = 128
WINDOW = 128
KW = 768
SCALE = 0.08838834764831843
CHUNK = SQ // N_DEV
HALF = D_MODEL // 2

CHUNK_ORDER = [0, 3, 1, 2]


def _body(x_ref, wq_ref, k_hbm, v_hbm, wo_ref, out_ref,
          q_ref, kbuf, vbuf, sc_ref, ctx_ref, p_ref,
          send_cw, send_ccw, recv_cw, recv_ccw,
          k_sems, v_sems,
          send_sems_cw, recv_sems_cw, send_sems_ccw, recv_sems_ccw):
    my = lax.axis_index("i")
    left = (my - 1) % N_DEV
    right = (my + 1) % N_DEV

    barrier_sem = pltpu.get_barrier_semaphore()
    for nbr in [left, right]:
        pl.semaphore_signal(barrier_sem, inc=1, device_id=(nbr,),
                            device_id_type=pl.DeviceIdType.MESH)

    def wstart(t):
        c = (my + CHUNK_ORDER[t]) % N_DEV
        return jnp.clip(c * CHUNK - WINDOW, 0, SKV - KW)

    def kv_dma(t, h, slot):
        start = wstart(t)
        return (
            pltpu.make_async_copy(k_hbm.at[pl.ds(start, KW), h, :],
                                  kbuf.at[slot], k_sems.at[slot]),
            pltpu.make_async_copy(v_hbm.at[pl.ds(start, KW), h, :],
                                  vbuf.at[slot], v_sems.at[slot]),
        )

    def compute_chunk(t):
        c = (my + CHUNK_ORDER[t]) % N_DEV
        qs = c * CHUNK
        start = wstart(t)
        q_ref[:, :] = jnp.dot(x_ref[0, pl.ds(qs, CHUNK), :], wq_ref[:, :],
                              preferred_element_type=jnp.float32)
        qi = qs + lax.broadcasted_iota(jnp.int32, (CHUNK, KW), 0)
        ki = start + lax.broadcasted_iota(jnp.int32, (CHUNK, KW), 1)
        keep = jnp.abs(qi - ki) <= WINDOW
        for h in range(HQ_PER):
            slot = h % 2
            if h + 1 < HQ_PER:
                for cp in kv_dma(t, h + 1, (h + 1) % 2):
                    cp.start()
            for cp in kv_dma(t, h, slot):
                cp.wait()
            q = q_ref[:, h * DH:(h + 1) * DH]
            s = lax.dot_general(
                q, kbuf[slot], (((1,), (1,)), ((), ())),
                preferred_element_type=jnp.float32) * SCALE
            sc_ref[:, :] = jnp.where(keep, s, jnp.float32(-1e9))
            s = sc_ref[:, :]
            m = jnp.max(s, axis=-1, keepdims=True)
            w = jnp.exp(s - m)
            w = w / jnp.sum(w, axis=-1, keepdims=True)
            ctx_ref[:, h * DH:(h + 1) * DH] = jnp.dot(
                w, vbuf[slot], preferred_element_type=jnp.float32)
        p_ref[pl.ds(qs, CHUNK), :] = jnp.dot(
            ctx_ref[:, :], wo_ref[:, :], preferred_element_type=jnp.float32)
        if t + 1 < N_DEV:
            for cp in kv_dma(t + 1, 0, 0):
                cp.start()

    def hop(step):
        cw = pltpu.make_async_remote_copy(
            src_ref=send_cw, dst_ref=recv_cw.at[step],
            send_sem=send_sems_cw.at[step], recv_sem=recv_sems_cw.at[step],
            device_id=(right,), device_id_type=pl.DeviceIdType.MESH,
        )
        ccw = pltpu.make_async_remote_copy(
            src_ref=send_ccw, dst_ref=recv_ccw.at[step],
            send_sem=send_sems_ccw.at[step], recv_sem=recv_sems_ccw.at[step],
            device_id=(left,), device_id_type=pl.DeviceIdType.MESH,
        )
        cw.start()
        ccw.start()
        return cw, ccw

    for cp in kv_dma(0, 0, 0):
        cp.start()
    compute_chunk(0)
    send_cw[:, :] = p_ref[pl.ds(my * CHUNK, CHUNK), :HALF].astype(jnp.bfloat16)
    send_ccw[:, :] = p_ref[pl.ds(my * CHUNK, CHUNK), HALF:].astype(jnp.bfloat16)
    pl.semaphore_wait(barrier_sem, 2)
    hops = hop(0)

    compute_chunk(1)
    compute_chunk(2)

    for s in range(N_DEV - 1):
        hops[0].wait()
        hops[1].wait()
        cw_idx = (my - s - 1) % N_DEV
        ccw_idx = (my + s + 1) % N_DEV
        acc_cw = (recv_cw[s].astype(jnp.float32)
                  + p_ref[pl.ds(cw_idx * CHUNK, CHUNK), :HALF])
        acc_ccw = (recv_ccw[s].astype(jnp.float32)
                   + p_ref[pl.ds(ccw_idx * CHUNK, CHUNK), HALF:])
        send_cw[:, :] = acc_cw.astype(jnp.bfloat16)
        send_ccw[:, :] = acc_ccw.astype(jnp.bfloat16)
        if s < N_DEV - 2:
            hops = hop(s + 1)
            if s == 0:
                compute_chunk(3)
        else:
            out_ref[0, pl.ds(((my + 1) % N_DEV) * CHUNK, CHUNK), :HALF] = acc_cw
            out_ref[0, pl.ds(((my - 1) % N_DEV) * CHUNK, CHUNK), HALF:] = acc_ccw

    for s in range(N_DEV - 1):
        c1, c2 = hop(N_DEV - 1 + s)
        c1.wait()
        c2.wait()
        cw_idx = (my - s) % N_DEV
        ccw_idx = (my + s) % N_DEV
        out_ref[0, pl.ds(cw_idx * CHUNK, CHUNK), :HALF] = (
            recv_cw[N_DEV - 1 + s].astype(jnp.float32))
        out_ref[0, pl.ds(ccw_idx * CHUNK, CHUNK), HALF:] = (
            recv_ccw[N_DEV - 1 + s].astype(jnp.float32))
        if s < N_DEV - 2:
            send_cw[:, :] = recv_cw[N_DEV - 1 + s]
            send_ccw[:, :] = recv_ccw[N_DEV - 1 + s]


def kernel(x, Wq, K_ext, V_ext, Wo):
    my = lax.axis_index("i")
    k_sh = lax.dynamic_slice_in_dim(K_ext[0], my * HQ_PER, HQ_PER, axis=1)
    v_sh = lax.dynamic_slice_in_dim(V_ext[0], my * HQ_PER, HQ_PER, axis=1)
    return pl.pallas_call(
        _body,
        out_shape=jax.ShapeDtypeStruct((1, SQ, D_MODEL), jnp.float32),
        in_specs=[
            pl.BlockSpec(memory_space=pltpu.VMEM),
            pl.BlockSpec(memory_space=pltpu.VMEM),
            pl.BlockSpec(memory_space=pltpu.MemorySpace.HBM),
            pl.BlockSpec(memory_space=pltpu.MemorySpace.HBM),
            pl.BlockSpec(memory_space=pltpu.VMEM),
        ],
        out_specs=pl.BlockSpec(memory_space=pltpu.VMEM),
        scratch_shapes=[
            pltpu.VMEM((CHUNK, D_MODEL), jnp.float32),
            pltpu.VMEM((2, KW, DH), jnp.float32),
            pltpu.VMEM((2, KW, DH), jnp.float32),
            pltpu.VMEM((CHUNK, KW), jnp.float32),
            pltpu.VMEM((CHUNK, D_MODEL), jnp.float32),
            pltpu.VMEM((SQ, D_MODEL), jnp.float32),
            pltpu.VMEM((CHUNK, HALF), jnp.bfloat16),
            pltpu.VMEM((CHUNK, HALF), jnp.bfloat16),
            pltpu.VMEM((2 * (N_DEV - 1), CHUNK, HALF), jnp.bfloat16),
            pltpu.VMEM((2 * (N_DEV - 1), CHUNK, HALF), jnp.bfloat16),
            pltpu.SemaphoreType.DMA((2,)),
            pltpu.SemaphoreType.DMA((2,)),
            pltpu.SemaphoreType.DMA((2 * (N_DEV - 1),)),
            pltpu.SemaphoreType.DMA((2 * (N_DEV - 1),)),
            pltpu.SemaphoreType.DMA((2 * (N_DEV - 1),)),
            pltpu.SemaphoreType.DMA((2 * (N_DEV - 1),)),
        ],
        compiler_params=pltpu.CompilerParams(
            collective_id=0,
            vmem_limit_bytes=64 * 1024 * 1024,
        ),
    )(x, Wq, k_sh, v_sh, Wo)
